# Initial kernel scaffold; baseline (speedup 1.0000x reference)
import jax
import jax.numpy as jnp
from jax.experimental import pallas as pl


def kernel(user_id, car_id, interaction, user_table, car_table, W, b):
    raise NotImplementedError("write your pallas kernel here")


# speedup vs baseline: 1.4159x; 1.4159x over previous
import functools

import jax
import jax.numpy as jnp
from jax import lax
from jax.experimental import pallas as pl
from jax.experimental.pallas import tpu as pltpu
from jax.experimental.pallas import tpu_sc as plsc

BATCH = 16384
EMBED_DIM = 32

try:
    _INFO = plsc.get_sparse_core_info()
    _NC, _NS = _INFO.num_cores, _INFO.num_subcores
except Exception:
    _NC, _NS = 2, 16
_NW = _NC * _NS
_BPW = BATCH // _NW
_CHUNK = 128
_NCHUNK = _BPW // _CHUNK


def _body(uid_hbm, cid_hbm, ut_hbm, ct_hbm, w_hbm, out_hbm,
          idx_u, idx_c, rows_u, rows_c, w_v, out_v, sem):
    wid = lax.axis_index("s") * _NC + lax.axis_index("c")
    base = wid * _BPW

    pltpu.sync_copy(w_hbm, w_v)
    pltpu.sync_copy(uid_hbm.at[wid], idx_u)
    pltpu.sync_copy(cid_hbm.at[wid], idx_c)

    copies = []
    for j in range(_NCHUNK):
        copies.append(pltpu.async_copy(
            ut_hbm.at[idx_u.at[j]], rows_u.at[pl.ds(j * _CHUNK, _CHUNK)], sem))
        copies.append(pltpu.async_copy(
            ct_hbm.at[idx_c.at[j]], rows_c.at[pl.ds(j * _CHUNK, _CHUNK)], sem))
    for c in copies:
        c.wait()

    lanes = lax.iota(jnp.int32, 16)

    wsl = [w_v[pl.ds(k * 16, 16)] for k in range(4)]
    bias = w_v[pl.ds(56, 16)][8]

    def blk_body(blk, carry):
        rowv = lanes + blk * 16
        acc = jnp.zeros((16,), jnp.float32) + bias
        for d in range(EMBED_DIM):
            colv = jnp.full((16,), d, jnp.int32)
            acc = acc + plsc.load_gather(rows_u, [rowv, colv]) * wsl[d // 16][d % 16]
            dc = EMBED_DIM + d
            acc = acc + plsc.load_gather(rows_c, [rowv, colv]) * wsl[dc // 16][dc % 16]
        out_v[pl.ds(blk * 16, 16)] = acc
        return carry

    lax.fori_loop(0, _BPW // 16, blk_body, 0)

    pltpu.sync_copy(out_v, out_hbm.at[pl.ds(base, _BPW)])


@functools.partial(jax.jit, static_argnames=())
def _run(uid3d, cid3d, user_table, car_table, w_ext):
    mesh = plsc.VectorSubcoreMesh(core_axis_name="c", subcore_axis_name="s")
    k = pl.kernel(
        _body,
        mesh=mesh,
        out_type=jax.ShapeDtypeStruct((BATCH,), jnp.float32),
        compiler_params=pltpu.CompilerParams(
            needs_layout_passes=False, use_tc_tiling_on_sc=False),
        scratch_types=[
            pltpu.VMEM((_NCHUNK, _CHUNK), jnp.int32),
            pltpu.VMEM((_NCHUNK, _CHUNK), jnp.int32),
            pltpu.VMEM((_BPW, EMBED_DIM), jnp.float32),
            pltpu.VMEM((_BPW, EMBED_DIM), jnp.float32),
            pltpu.VMEM((72,), jnp.float32),
            pltpu.VMEM((_BPW,), jnp.float32),
            pltpu.SemaphoreType.DMA,
        ],
    )
    return k(uid3d, cid3d, user_table, car_table, w_ext)


def kernel(user_id, car_id, interaction, user_table, car_table, W, b):
    del interaction
    uid3d = user_id.reshape(_NW, _NCHUNK, _CHUNK)
    cid3d = car_id.reshape(_NW, _NCHUNK, _CHUNK)
    w_ext = jnp.concatenate(
        [W[:, 0], b, jnp.zeros((7,), jnp.float32)])
    out = _run(uid3d, cid3d, user_table, car_table, w_ext)
    return out.reshape(BATCH, 1)


# speedup vs baseline: 3.3387x; 2.3580x over previous
import functools

import jax
import jax.numpy as jnp
from jax import lax
from jax.experimental import pallas as pl
from jax.experimental.pallas import tpu as pltpu
from jax.experimental.pallas import tpu_sc as plsc

BATCH = 16384
EMBED_DIM = 32
NROWS = 100000

try:
    _INFO = plsc.get_sparse_core_info()
    _NC, _NS = _INFO.num_cores, _INFO.num_subcores
except Exception:
    _NC, _NS = 2, 16
_NW = _NC * _NS
_BPW = BATCH // _NW
_CHUNK = 128
_NCHUNK = _BPW // _CHUNK

_BN = 2048


def _proj_body(b_ref, utT_ref, ctT_ref, wu_ref, wc_ref, pu_ref, pc_ref):
    u = utT_ref[...]
    c = ctT_ref[...]
    pu_ref[...] = jnp.sum(u * wu_ref[...], axis=0) + b_ref[0]
    pc_ref[...] = jnp.sum(c * wc_ref[...], axis=0)


def _gather_body(pu_hbm, pc_hbm, uid_hbm, cid_hbm, out_hbm,
                 idx_u, idx_c, pu_v, pc_v, out_v, sem):
    wid = lax.axis_index("s") * _NC + lax.axis_index("c")
    base = wid * _BPW

    pltpu.sync_copy(uid_hbm.at[wid], idx_u)
    pltpu.sync_copy(cid_hbm.at[wid], idx_c)

    copies = []
    for j in range(_NCHUNK):
        sl = pl.ds(j * _CHUNK, _CHUNK)
        copies.append(pltpu.async_copy(pu_hbm.at[idx_u.at[j]], pu_v.at[sl], sem))
        copies.append(pltpu.async_copy(pc_hbm.at[idx_c.at[j]], pc_v.at[sl], sem))
    for cpy in copies:
        cpy.wait()

    for k in range(_BPW // 16):
        sl = pl.ds(k * 16, 16)
        out_v[sl] = pu_v[sl] + pc_v[sl]

    pltpu.sync_copy(out_v, out_hbm.at[pl.ds(base, _BPW)])


@jax.jit
def _run(uid3d, cid3d, utT, ctT, wu, wc, b):
    grid = (NROWS + _BN - 1) // _BN
    proj_u, proj_c = pl.pallas_call(
        _proj_body,
        grid=(grid,),
        in_specs=[
            pl.BlockSpec(memory_space=pltpu.SMEM),
            pl.BlockSpec((EMBED_DIM, _BN), lambda i: (0, i)),
            pl.BlockSpec((EMBED_DIM, _BN), lambda i: (0, i)),
            pl.BlockSpec((EMBED_DIM, 1), lambda i: (0, 0)),
            pl.BlockSpec((EMBED_DIM, 1), lambda i: (0, 0)),
        ],
        out_specs=[
            pl.BlockSpec((_BN,), lambda i: (i,)),
            pl.BlockSpec((_BN,), lambda i: (i,)),
        ],
        out_shape=[
            jax.ShapeDtypeStruct((NROWS,), jnp.float32),
            jax.ShapeDtypeStruct((NROWS,), jnp.float32),
        ],
    )(b, utT, ctT, wu, wc)

    mesh = plsc.VectorSubcoreMesh(core_axis_name="c", subcore_axis_name="s")
    k = pl.kernel(
        _gather_body,
        mesh=mesh,
        out_type=jax.ShapeDtypeStruct((BATCH,), jnp.float32),
        compiler_params=pltpu.CompilerParams(
            needs_layout_passes=False, use_tc_tiling_on_sc=False),
        scratch_types=[
            pltpu.VMEM((_NCHUNK, _CHUNK), jnp.int32),
            pltpu.VMEM((_NCHUNK, _CHUNK), jnp.int32),
            pltpu.VMEM((_BPW,), jnp.float32),
            pltpu.VMEM((_BPW,), jnp.float32),
            pltpu.VMEM((_BPW,), jnp.float32),
            pltpu.SemaphoreType.DMA,
        ],
    )
    return k(proj_u, proj_c, uid3d, cid3d)


def kernel(user_id, car_id, interaction, user_table, car_table, W, b):
    del interaction
    uid3d = user_id.reshape(_NW, _NCHUNK, _CHUNK)
    cid3d = car_id.reshape(_NW, _NCHUNK, _CHUNK)
    out = _run(uid3d, cid3d, user_table.T, car_table.T,
               W[:EMBED_DIM], W[EMBED_DIM:], b)
    return out.reshape(BATCH, 1)


# speedup vs baseline: 5.4874x; 1.6436x over previous
import functools

import jax
import jax.numpy as jnp
from jax import lax
from jax.experimental import pallas as pl
from jax.experimental.pallas import tpu as pltpu
from jax.experimental.pallas import tpu_sc as plsc

BATCH = 16384
EMBED_DIM = 32
NROWS = 100000

try:
    _INFO = plsc.get_sparse_core_info()
    _NC, _NS = _INFO.num_cores, _INFO.num_subcores
except Exception:
    _NC, _NS = 2, 16
_NW = _NC * _NS
_BPW = BATCH // _NW
_CHUNK = 128
_NCHUNK = _BPW // _CHUNK

_BN = 25600


def _proj_body(b_ref, utT_ref, ctT_ref, wu_ref, wc_ref, pu_ref, pc_ref):
    u = utT_ref[...]
    c = ctT_ref[...]
    pu_ref[...] = jnp.sum(u * wu_ref[...], axis=0) + b_ref[0]
    pc_ref[...] = jnp.sum(c * wc_ref[...], axis=0)


def _gather_body(pu_hbm, pc_hbm, uid_hbm, cid_hbm, out_hbm,
                 idx_u, idx_c, pu_v, pc_v, out_v, sem):
    wid = lax.axis_index("s") * _NC + lax.axis_index("c")
    base = wid * _BPW

    pltpu.sync_copy(uid_hbm.at[wid], idx_u)
    pltpu.sync_copy(cid_hbm.at[wid], idx_c)

    copies = []
    for j in range(_NCHUNK):
        sl = pl.ds(j * _CHUNK, _CHUNK)
        copies.append(pltpu.async_copy(pu_hbm.at[idx_u.at[j]], pu_v.at[sl], sem))
        copies.append(pltpu.async_copy(pc_hbm.at[idx_c.at[j]], pc_v.at[sl], sem))
    for cpy in copies:
        cpy.wait()

    for k in range(_BPW // 16):
        sl = pl.ds(k * 16, 16)
        out_v[sl] = pu_v[sl] + pc_v[sl]

    pltpu.sync_copy(out_v, out_hbm.at[pl.ds(base, _BPW)])


@jax.jit
def _run(uid3d, cid3d, utT, ctT, wu, wc, b):
    grid = (NROWS + _BN - 1) // _BN
    proj_u, proj_c = pl.pallas_call(
        _proj_body,
        grid=(grid,),
        in_specs=[
            pl.BlockSpec(memory_space=pltpu.SMEM),
            pl.BlockSpec((EMBED_DIM, _BN), lambda i: (0, i)),
            pl.BlockSpec((EMBED_DIM, _BN), lambda i: (0, i)),
            pl.BlockSpec((EMBED_DIM, 1), lambda i: (0, 0)),
            pl.BlockSpec((EMBED_DIM, 1), lambda i: (0, 0)),
        ],
        out_specs=[
            pl.BlockSpec((_BN,), lambda i: (i,)),
            pl.BlockSpec((_BN,), lambda i: (i,)),
        ],
        out_shape=[
            jax.ShapeDtypeStruct((NROWS,), jnp.float32),
            jax.ShapeDtypeStruct((NROWS,), jnp.float32),
        ],
    )(b, utT, ctT, wu, wc)

    mesh = plsc.VectorSubcoreMesh(core_axis_name="c", subcore_axis_name="s")
    k = pl.kernel(
        _gather_body,
        mesh=mesh,
        out_type=jax.ShapeDtypeStruct((BATCH,), jnp.float32),
        compiler_params=pltpu.CompilerParams(
            needs_layout_passes=False, use_tc_tiling_on_sc=False),
        scratch_types=[
            pltpu.VMEM((_NCHUNK, _CHUNK), jnp.int32),
            pltpu.VMEM((_NCHUNK, _CHUNK), jnp.int32),
            pltpu.VMEM((_BPW,), jnp.float32),
            pltpu.VMEM((_BPW,), jnp.float32),
            pltpu.VMEM((_BPW,), jnp.float32),
            pltpu.SemaphoreType.DMA,
        ],
    )
    return k(proj_u, proj_c, uid3d, cid3d)


def kernel(user_id, car_id, interaction, user_table, car_table, W, b):
    del interaction
    uid3d = user_id.reshape(_NW, _NCHUNK, _CHUNK)
    cid3d = car_id.reshape(_NW, _NCHUNK, _CHUNK)
    out = _run(uid3d, cid3d, user_table.T, car_table.T,
               W[:EMBED_DIM], W[EMBED_DIM:], b)
    return out.reshape(BATCH, 1)
